# baseline (speedup 1.0000x reference)
import functools

import jax
import jax.numpy as jnp
from jax import lax
from jax.experimental import pallas as pl
from jax.experimental.pallas import tpu as pltpu
from jax.experimental.pallas import tpu_sc as plsc

EPS = 1e-5

_NCHUNK = 5
_GCHUNK = 128
_NBUF = 5
_PREF = 2


def _make_sc_gather(ntok, v, d):
    info = plsc.get_sparse_core_info()
    nw = info.num_cores * info.num_subcores
    assert ntok % (nw * _GCHUNK) == 0
    tw = ntok // nw
    ng = tw // _GCHUNK
    assert ng % _NBUF == 0
    mesh = plsc.VectorSubcoreMesh(core_axis_name="c", subcore_axis_name="s")

    @functools.partial(
        pl.kernel,
        mesh=mesh,
        out_type=jax.ShapeDtypeStruct((ntok, d), jnp.float32),
        scratch_types=[
            pltpu.VMEM((tw,), jnp.int32),
            pltpu.VMEM((_NBUF, _GCHUNK, d), jnp.float32),
            pltpu.SemaphoreType.DMA,
            pltpu.SemaphoreType.DMA,
        ],
    )
    def gather_kernel(tbl_hbm, ids_hbm, out_hbm, idx_v, rows_v, gsem, wsem):
        wid = lax.axis_index("s") * info.num_cores + lax.axis_index("c")
        base = wid * tw
        pltpu.sync_copy(ids_hbm.at[pl.ds(base, tw)], idx_v)

        def start_gather(g, b):
            pltpu.async_copy(
                tbl_hbm.at[idx_v.at[pl.ds(g * _GCHUNK, _GCHUNK)]],
                rows_v.at[b],
                gsem,
            )

        def start_write(g, b):
            pltpu.async_copy(
                rows_v.at[b],
                out_hbm.at[pl.ds(base + g * _GCHUNK, _GCHUNK)],
                wsem,
            )

        def wait_gather(b):
            pltpu.make_async_copy(tbl_hbm.at[idx_v.at[pl.ds(0, _GCHUNK)]],
                                  rows_v.at[b], gsem).wait()

        def wait_write(b):
            pltpu.make_async_copy(rows_v.at[b],
                                  out_hbm.at[pl.ds(base, _GCHUNK)], wsem).wait()

        for b in range(_PREF):
            start_gather(b, b)

        def outer(i, _):
            g0 = i * _NBUF
            for b in range(_NBUF):
                g = g0 + b
                wait_gather(b)
                start_write(g, b)
                j = g + _PREF
                bj = (b + _PREF) % _NBUF

                @pl.when(jnp.logical_and(j < ng, j >= _NBUF))
                def _():
                    wait_write(bj)
                    start_gather(j, bj)

                @pl.when(jnp.logical_and(j < ng, j < _NBUF))
                def _():
                    start_gather(j, bj)

            return 0

        lax.fori_loop(0, ng // _NBUF, outer, 0)
        for _ in range(min(_NBUF, ng)):
            wait_write(0)

    return gather_kernel


def _ln_first_body(raw_ref, gamma_ref, beta_ref, out_ref):
    x = raw_ref[...]
    mean = jnp.mean(x, axis=-1, keepdims=True)
    xc = x - mean
    var = jnp.mean(xc * xc, axis=-1, keepdims=True)
    inv = lax.rsqrt(var + EPS)
    out_ref[...] = xc * inv * gamma_ref[...] + beta_ref[...]


def _ln_next_body(acc_ref, raw_ref, gamma_ref, beta_ref, out_ref):
    _ln_first_body(raw_ref, gamma_ref, beta_ref, out_ref)


def _ln_chunk(acc, raw, gamma, beta, chunk_idx, ntok, block_rows):
    nc, d = raw.shape
    assert nc % block_rows == 0 and (chunk_idx * nc) % block_rows == 0
    nb = nc // block_rows
    blk0 = chunk_idx * nc // block_rows
    grid = (nb,)
    raw_spec = pl.BlockSpec((block_rows, d), lambda i: (i, 0))
    gb_spec = pl.BlockSpec((1, d), lambda i: (0, 0))
    out_spec = pl.BlockSpec((block_rows, d), lambda i: (blk0 + i, 0))
    out_shape = jax.ShapeDtypeStruct((ntok, d), jnp.float32)
    g2 = gamma.reshape(1, d)
    b2 = beta.reshape(1, d)
    if acc is None:
        return pl.pallas_call(
            _ln_first_body,
            grid=grid,
            in_specs=[raw_spec, gb_spec, gb_spec],
            out_specs=out_spec,
            out_shape=out_shape,
        )(raw, g2, b2)
    return pl.pallas_call(
        _ln_next_body,
        grid=grid,
        in_specs=[pl.BlockSpec(memory_space=pl.ANY),
                  raw_spec, gb_spec, gb_spec],
        out_specs=out_spec,
        out_shape=out_shape,
        input_output_aliases={0: 0},
    )(acc, raw, g2, b2)


def kernel(input_ids, table, gamma, beta):
    b, s = input_ids.shape
    v, d = table.shape
    ntok = b * s
    assert ntok % _NCHUNK == 0
    nc = ntok // _NCHUNK
    ids_flat = input_ids.reshape(-1).astype(jnp.int32)
    sc_gather = _make_sc_gather(nc, v, d)

    raws = [sc_gather(table, ids_flat[c * nc:(c + 1) * nc])
            for c in range(_NCHUNK)]
    acc = None
    for c in range(_NCHUNK):
        acc = _ln_chunk(acc, raws[c], gamma, beta, c, ntok, block_rows=8192)
    return acc.reshape(b, s, d)
